# baseline (speedup 1.0000x reference)
import functools

import jax
import jax.numpy as jnp
from jax import lax
from jax.experimental import pallas as pl
from jax.experimental.pallas import tpu as pltpu
from jax.experimental.pallas import tpu_sc as plsc

N = 10000
E = 320000
D = 128

NC = 2
NS = 16
NW = NC * NS

EDGES_PER_TILE = E // NS
CHUNK = 100
NCHUNKS = EDGES_PER_TILE // CHUNK
BLK = 80
NBLK = N // BLK
SLAB = 40
NSLABS = NCHUNKS // SLAB
NBUF = 3

_mesh = plsc.VectorSubcoreMesh(core_axis_name="c", subcore_axis_name="s")
_sc_params = pltpu.CompilerParams(needs_layout_passes=False)


@functools.partial(
    pl.kernel,
    out_type=jax.ShapeDtypeStruct((NW, N), jnp.float32),
    mesh=_mesh,
    scratch_types=[
        pltpu.VMEM((E // NW,), jnp.int32),
        pltpu.VMEM((N,), jnp.float32),
    ],
    compiler_params=_sc_params,
)
def _deg_kernel(dst_hbm, degp_hbm, dstv, degv):
    c = lax.axis_index("c")
    s = lax.axis_index("s")
    wid = s * NC + c
    pltpu.sync_copy(dst_hbm.at[wid], dstv)

    zeros = jnp.zeros((16,), jnp.float32)

    def zero_body(i, carry):
        degv[pl.ds(i * 16, 16)] = zeros
        return carry

    lax.fori_loop(0, N // 16, zero_body, 0)

    ones = jnp.ones((16,), jnp.float32)

    def add_body(i, carry):
        idx = dstv[pl.ds(i * 16, 16)]
        plsc.addupdate_scatter(degv, [idx], ones)
        return carry

    lax.fori_loop(0, (E // NW) // 16, add_body, 0)
    pltpu.sync_copy(degv, degp_hbm.at[wid])


def _dinv_col(degp):
    ones = jnp.ones((NW, 1), jnp.float32)
    deg = lax.dot_general(degp, ones, (((0,), (0,)), ((), ())),
                          preferred_element_type=jnp.float32) + 1.0
    return lax.rsqrt(deg)


def _front_body(x_ref, wa_ref, wc_ref, degp_ref, ya_ref, yc_ref):
    dv = _dinv_col(degp_ref[...])
    xv = x_ref[...]
    f32 = jnp.float32
    ya_ref[...] = jnp.dot(xv, wa_ref[...], preferred_element_type=f32) * dv
    yc_ref[...] = jnp.dot(xv, wc_ref[...], preferred_element_type=f32) * dv


_front_call = pl.pallas_call(
    _front_body,
    out_shape=[
        jax.ShapeDtypeStruct((N, D), jnp.float32),
        jax.ShapeDtypeStruct((N, D), jnp.float32),
    ],
)


@functools.partial(
    pl.kernel,
    out_type=[
        jax.ShapeDtypeStruct((N, D), jnp.float32),
        jax.ShapeDtypeStruct((N, D), jnp.float32),
    ],
    mesh=_mesh,
    scratch_types=[
        pltpu.VMEM((SLAB, CHUNK), jnp.int32),
        pltpu.VMEM((SLAB, CHUNK), jnp.int32),
        pltpu.VMEM((CHUNK, D), jnp.float32),
        pltpu.VMEM((CHUNK, D), jnp.float32),
        pltpu.VMEM((CHUNK, D), jnp.float32),
        pltpu.VMEM_SHARED((N, D), jnp.float32),
        pltpu.SemaphoreType.DMA,
        pltpu.SemaphoreType.DMA,
        pltpu.SemaphoreType.DMA,
        pltpu.SemaphoreType.DMA,
        pltpu.SemaphoreType.DMA,
        pltpu.SemaphoreType.DMA,
    ],
    compiler_params=_sc_params,
)
def _edge_kernel(ya_hbm, yc_hbm, ei_hbm, acca_hbm, accc_hbm,
                 srcv, dstv, rows0, rows1, rows2, accs,
                 g0, g1, g2, s0, s1, s2):
    c = lax.axis_index("c")
    s = lax.axis_index("s")
    bufs = (rows0, rows1, rows2)
    gsems = (g0, g1, g2)
    ssems = (s0, s1, s2)

    def run(y_hbm, out_hbm):
        for j in range((NBLK + NS - 1) // NS):
            g = j * NS + s

            @pl.when(g < NBLK)
            def _():
                pltpu.sync_copy(y_hbm.at[pl.ds(g * BLK, BLK)],
                                accs.at[pl.ds(g * BLK, BLK)])
        plsc.subcore_barrier()

        def gstart(l, b):
            pltpu.async_copy(y_hbm.at[srcv.at[l]], bufs[b], gsems[b])

        def gwait(b):
            pltpu.make_async_copy(y_hbm.at[srcv.at[0]], bufs[b], gsems[b]).wait()

        def sstart(l, b):
            pltpu.async_copy(bufs[b], accs.at[dstv.at[l]], ssems[b], add=True)

        def swait(b):
            pltpu.make_async_copy(bufs[b], accs.at[dstv.at[0]], ssems[b]).wait()

        for slab in range(NSLABS):
            ph = (slab * SLAB) % NBUF
            pltpu.sync_copy(ei_hbm.at[0, s, pl.ds(slab * SLAB, SLAB)], srcv)
            pltpu.sync_copy(ei_hbm.at[1, s, pl.ds(slab * SLAB, SLAB)], dstv)
            for l in range(NBUF - 1):
                gstart(l, (l + ph) % NBUF)

            def group(m, carry):
                l0 = m * NBUF
                for k in range(NBUF):
                    b = (k + ph) % NBUF
                    gwait(b)
                    sstart(l0 + k, b)
                    if k == 0:
                        @pl.when(m > 0)
                        def _():
                            swait((ph - 1) % NBUF)
                    else:
                        swait((k - 1 + ph) % NBUF)
                    if k == 0:
                        gstart(l0 + k + NBUF - 1, (b + NBUF - 1) % NBUF)
                    else:
                        @pl.when(l0 + k + NBUF - 1 < SLAB)
                        def _():
                            gstart(l0 + k + NBUF - 1, (b + NBUF - 1) % NBUF)
                return carry

            ngroups = SLAB // NBUF
            lax.fori_loop(0, ngroups, group, 0)
            for l in range(ngroups * NBUF, SLAB):
                b = (l + ph) % NBUF
                gwait(b)
                sstart(l, b)
                swait((b + NBUF - 1) % NBUF)
            swait((SLAB - 1 + ph) % NBUF)
        plsc.subcore_barrier()
        for j in range((NBLK + NS - 1) // NS):
            g = j * NS + s

            @pl.when(g < NBLK)
            def _():
                pltpu.sync_copy(accs.at[pl.ds(g * BLK, BLK)],
                                out_hbm.at[pl.ds(g * BLK, BLK)])

    @pl.when(c == 0)
    def _():
        run(ya_hbm, acca_hbm)

    @pl.when(c == 1)
    def _():
        run(yc_hbm, accc_hbm)


def _softplus(v):
    return jnp.maximum(v, 0.0) + jnp.log1p(jnp.exp(-jnp.abs(v)))


def _epi_body(acca_ref, accc_ref, degp_ref, x_ref,
              bca_ref, w1a_ref, b1a_ref, w2a_ref, b2a_ref, w3a_ref, b3a_ref,
              bcc_ref, w1c_ref, b1c_ref, w2c_ref, b2c_ref, w3c_ref, b3c_ref,
              conc_ref, probs_ref, val_ref):
    dv = _dinv_col(degp_ref[...])
    xv = x_ref[...]
    f32 = jnp.float32

    ha = jnp.maximum(dv * acca_ref[...] + bca_ref[...], 0.0)
    xa = ha + xv
    t = jnp.maximum(jnp.dot(xa, w1a_ref[...], preferred_element_type=f32)
                    + b1a_ref[...], 0.0)
    t = jnp.maximum(jnp.dot(t, w2a_ref[...], preferred_element_type=f32)
                    + b2a_ref[...], 0.0)
    ao = (lax.dot_general(w3a_ref[...], t, (((0,), (1,)), ((), ())),
                          preferred_element_type=f32)
          + b3a_ref[...][:, None])
    conc_ref[...] = _softplus(ao[0:1, :]) + 1e-20
    p2 = _softplus(ao[1:2, :])
    probs_ref[...] = p2 / jnp.sum(p2)

    hc = jnp.maximum(dv * accc_ref[...] + bcc_ref[...], 0.0)
    xc = jnp.sum(hc + xv, axis=0, keepdims=True)
    u = jnp.maximum(jnp.dot(xc, w1c_ref[...], preferred_element_type=f32)
                    + b1c_ref[...], 0.0)
    u = jnp.maximum(jnp.dot(u, w2c_ref[...], preferred_element_type=f32)
                    + b2c_ref[...], 0.0)
    val_ref[...] = jnp.dot(u, w3c_ref[...], preferred_element_type=f32) + b3c_ref[...]


_epi_call = pl.pallas_call(
    _epi_body,
    out_shape=[
        jax.ShapeDtypeStruct((1, N), jnp.float32),
        jax.ShapeDtypeStruct((1, N), jnp.float32),
        jax.ShapeDtypeStruct((1, 2), jnp.float32),
    ],
)


def kernel(x, edge_index, Wc_a, bc_a, W1_a, b1_a, W2_a, b2_a, W3_a, b3_a,
           Wc_c, bc_c, W1_c, b1_c, W2_c, b2_c, W3_c, b3_c):
    ei4 = edge_index.reshape(2, NS, NCHUNKS, CHUNK)
    degp = _deg_kernel(edge_index[1].reshape(NW, E // NW))
    ya, yc = _front_call(x, Wc_a, Wc_c, degp)

    acca, accc = _edge_kernel(ya, yc, ei4)

    conc, probs, value = _epi_call(
        acca, accc, degp, x,
        bc_a, W1_a, b1_a, W2_a, b2_a, W3_a, b3_a,
        bc_c, W1_c, b1_c, W2_c, b2_c, W3_c, b3_c,
    )
    return conc.reshape(-1), value.reshape(-1), probs.reshape(-1)


# speedup vs baseline: 1.0307x; 1.0307x over previous
import functools

import jax
import jax.numpy as jnp
from jax import lax
from jax.experimental import pallas as pl
from jax.experimental.pallas import tpu as pltpu
from jax.experimental.pallas import tpu_sc as plsc

N = 10000
E = 320000
D = 128

NC = 2
NS = 16
NW = NC * NS

EDGES_PER_TILE = E // NS
CHUNK = 100
NCHUNKS = EDGES_PER_TILE // CHUNK
BLK = 80
NBLK = N // BLK
SLAB = 40
NSLABS = NCHUNKS // SLAB
NBUF = 3

_mesh = plsc.VectorSubcoreMesh(core_axis_name="c", subcore_axis_name="s")
_sc_params = pltpu.CompilerParams(needs_layout_passes=False)


@functools.partial(
    pl.kernel,
    out_type=jax.ShapeDtypeStruct((NW, N), jnp.float32),
    mesh=_mesh,
    scratch_types=[
        pltpu.VMEM((104, CHUNK), jnp.int32),
        pltpu.VMEM((N,), jnp.float32),
    ],
    compiler_params=_sc_params,
)
def _deg_kernel(ei_hbm, degp_hbm, dstv, degv):
    c = lax.axis_index("c")
    s = lax.axis_index("s")
    wid = s * NC + c
    half = wid % NC
    pltpu.sync_copy(ei_hbm.at[1, wid // NC, pl.ds(half * 96, 104)], dstv)

    zeros = jnp.zeros((16,), jnp.float32)

    def zero_body(i, carry):
        degv[pl.ds(i * 16, 16)] = zeros
        return carry

    lax.fori_loop(0, N // 16, zero_body, 0)

    ones = jnp.ones((16,), jnp.float32)
    lanes = lax.iota(jnp.int32, 16)

    def add_body_from(base):
        def add_body(i, carry):
            flat = base + i * 16 + lanes
            r = flat // CHUNK
            col = flat - r * CHUNK
            idx = plsc.load_gather(dstv, [r, col])
            plsc.addupdate_scatter(degv, [idx], ones)
            return carry
        return add_body

    @pl.when(half == 0)
    def _():
        lax.fori_loop(0, (104 * CHUNK) // 16, add_body_from(0), 0)

    @pl.when(half == 1)
    def _():
        lax.fori_loop(0, (96 * CHUNK) // 16, add_body_from(8 * CHUNK), 0)

    pltpu.sync_copy(degv, degp_hbm.at[wid])


def _dinv_col(degp):
    ones = jnp.ones((NW, 1), jnp.float32)
    deg = lax.dot_general(degp, ones, (((0,), (0,)), ((), ())),
                          preferred_element_type=jnp.float32) + 1.0
    return lax.rsqrt(deg)


def _front_body(x_ref, wa_ref, wc_ref, degp_ref, ya_ref, yc_ref):
    dv = _dinv_col(degp_ref[...])
    xv = x_ref[...]
    f32 = jnp.float32
    ya_ref[...] = jnp.dot(xv, wa_ref[...], preferred_element_type=f32) * dv
    yc_ref[...] = jnp.dot(xv, wc_ref[...], preferred_element_type=f32) * dv


_front_call = pl.pallas_call(
    _front_body,
    out_shape=[
        jax.ShapeDtypeStruct((N, D), jnp.float32),
        jax.ShapeDtypeStruct((N, D), jnp.float32),
    ],
)


@functools.partial(
    pl.kernel,
    out_type=[
        jax.ShapeDtypeStruct((N, D), jnp.float32),
        jax.ShapeDtypeStruct((N, D), jnp.float32),
    ],
    mesh=_mesh,
    scratch_types=[
        pltpu.VMEM((SLAB, CHUNK), jnp.int32),
        pltpu.VMEM((SLAB, CHUNK), jnp.int32),
        pltpu.VMEM((CHUNK, D), jnp.float32),
        pltpu.VMEM((CHUNK, D), jnp.float32),
        pltpu.VMEM((CHUNK, D), jnp.float32),
        pltpu.VMEM_SHARED((N, D), jnp.float32),
        pltpu.SemaphoreType.DMA,
        pltpu.SemaphoreType.DMA,
        pltpu.SemaphoreType.DMA,
        pltpu.SemaphoreType.DMA,
        pltpu.SemaphoreType.DMA,
        pltpu.SemaphoreType.DMA,
    ],
    compiler_params=_sc_params,
)
def _edge_kernel(ya_hbm, yc_hbm, ei_hbm, acca_hbm, accc_hbm,
                 srcv, dstv, rows0, rows1, rows2, accs,
                 g0, g1, g2, s0, s1, s2):
    c = lax.axis_index("c")
    s = lax.axis_index("s")
    bufs = (rows0, rows1, rows2)
    gsems = (g0, g1, g2)
    ssems = (s0, s1, s2)

    def run(y_hbm, out_hbm):
        for j in range((NBLK + NS - 1) // NS):
            g = j * NS + s

            @pl.when(g < NBLK)
            def _():
                pltpu.sync_copy(y_hbm.at[pl.ds(g * BLK, BLK)],
                                accs.at[pl.ds(g * BLK, BLK)])
        plsc.subcore_barrier()

        def gstart(l, b):
            pltpu.async_copy(y_hbm.at[srcv.at[l]], bufs[b], gsems[b])

        def gwait(b):
            pltpu.make_async_copy(y_hbm.at[srcv.at[0]], bufs[b], gsems[b]).wait()

        def sstart(l, b):
            pltpu.async_copy(bufs[b], accs.at[dstv.at[l]], ssems[b], add=True)

        def swait(b):
            pltpu.make_async_copy(bufs[b], accs.at[dstv.at[0]], ssems[b]).wait()

        for slab in range(NSLABS):
            ph = (slab * SLAB) % NBUF
            pltpu.sync_copy(ei_hbm.at[0, s, pl.ds(slab * SLAB, SLAB)], srcv)
            pltpu.sync_copy(ei_hbm.at[1, s, pl.ds(slab * SLAB, SLAB)], dstv)
            for l in range(NBUF - 1):
                gstart(l, (l + ph) % NBUF)

            def group(m, carry):
                l0 = m * NBUF
                for k in range(NBUF):
                    b = (k + ph) % NBUF
                    gwait(b)
                    sstart(l0 + k, b)
                    if k == 0:
                        @pl.when(m > 0)
                        def _():
                            swait((ph - 1) % NBUF)
                    else:
                        swait((k - 1 + ph) % NBUF)
                    if k == 0:
                        gstart(l0 + k + NBUF - 1, (b + NBUF - 1) % NBUF)
                    else:
                        @pl.when(l0 + k + NBUF - 1 < SLAB)
                        def _():
                            gstart(l0 + k + NBUF - 1, (b + NBUF - 1) % NBUF)
                return carry

            ngroups = SLAB // NBUF
            lax.fori_loop(0, ngroups, group, 0)
            for l in range(ngroups * NBUF, SLAB):
                b = (l + ph) % NBUF
                gwait(b)
                sstart(l, b)
                swait((b + NBUF - 1) % NBUF)
            swait((SLAB - 1 + ph) % NBUF)
        plsc.subcore_barrier()
        for j in range((NBLK + NS - 1) // NS):
            g = j * NS + s

            @pl.when(g < NBLK)
            def _():
                pltpu.sync_copy(accs.at[pl.ds(g * BLK, BLK)],
                                out_hbm.at[pl.ds(g * BLK, BLK)])

    @pl.when(c == 0)
    def _():
        run(ya_hbm, acca_hbm)

    @pl.when(c == 1)
    def _():
        run(yc_hbm, accc_hbm)


def _softplus(v):
    return jnp.maximum(v, 0.0) + jnp.log1p(jnp.exp(-jnp.abs(v)))


def _epi_body(acca_ref, accc_ref, degp_ref, x_ref,
              bca_ref, w1a_ref, b1a_ref, w2a_ref, b2a_ref, w3a_ref, b3a_ref,
              bcc_ref, w1c_ref, b1c_ref, w2c_ref, b2c_ref, w3c_ref, b3c_ref,
              conc_ref, probs_ref, val_ref):
    dv = _dinv_col(degp_ref[...])
    xv = x_ref[...]
    f32 = jnp.float32

    ha = jnp.maximum(dv * acca_ref[...] + bca_ref[...], 0.0)
    xa = ha + xv
    t = jnp.maximum(jnp.dot(xa, w1a_ref[...], preferred_element_type=f32)
                    + b1a_ref[...], 0.0)
    t = jnp.maximum(jnp.dot(t, w2a_ref[...], preferred_element_type=f32)
                    + b2a_ref[...], 0.0)
    ao = (lax.dot_general(w3a_ref[...], t, (((0,), (1,)), ((), ())),
                          preferred_element_type=f32)
          + b3a_ref[...][:, None])
    conc_ref[...] = _softplus(ao[0:1, :]) + 1e-20
    p2 = _softplus(ao[1:2, :])
    probs_ref[...] = p2 / jnp.sum(p2)

    hc = jnp.maximum(dv * accc_ref[...] + bcc_ref[...], 0.0)
    xc = jnp.sum(hc + xv, axis=0, keepdims=True)
    u = jnp.maximum(jnp.dot(xc, w1c_ref[...], preferred_element_type=f32)
                    + b1c_ref[...], 0.0)
    u = jnp.maximum(jnp.dot(u, w2c_ref[...], preferred_element_type=f32)
                    + b2c_ref[...], 0.0)
    val_ref[...] = jnp.dot(u, w3c_ref[...], preferred_element_type=f32) + b3c_ref[...]


_epi_call = pl.pallas_call(
    _epi_body,
    out_shape=[
        jax.ShapeDtypeStruct((1, N), jnp.float32),
        jax.ShapeDtypeStruct((1, N), jnp.float32),
        jax.ShapeDtypeStruct((1, 2), jnp.float32),
    ],
)


def kernel(x, edge_index, Wc_a, bc_a, W1_a, b1_a, W2_a, b2_a, W3_a, b3_a,
           Wc_c, bc_c, W1_c, b1_c, W2_c, b2_c, W3_c, b3_c):
    ei4 = edge_index.reshape(2, NS, NCHUNKS, CHUNK)
    degp = _deg_kernel(ei4)
    ya, yc = _front_call(x, Wc_a, Wc_c, degp)

    acca, accc = _edge_kernel(ya, yc, ei4)

    conc, probs, value = _epi_call(
        acca, accc, degp, x,
        bc_a, W1_a, b1_a, W2_a, b2_a, W3_a, b3_a,
        bc_c, W1_c, b1_c, W2_c, b2_c, W3_c, b3_c,
    )
    return conc.reshape(-1), value.reshape(-1), probs.reshape(-1)


# speedup vs baseline: 1.0547x; 1.0232x over previous
import functools

import jax
import jax.numpy as jnp
from jax import lax
from jax.experimental import pallas as pl
from jax.experimental.pallas import tpu as pltpu
from jax.experimental.pallas import tpu_sc as plsc

N = 10000
E = 320000
D = 128

NC = 2
NS = 16
NW = NC * NS

EDGES_PER_TILE = E // NS
CHUNK = 100
NCHUNKS = EDGES_PER_TILE // CHUNK
BLK = 80
NBLK = N // BLK
SLAB = 40
NSLABS = NCHUNKS // SLAB
NBUF = 3

_mesh = plsc.VectorSubcoreMesh(core_axis_name="c", subcore_axis_name="s")
_sc_params = pltpu.CompilerParams(needs_layout_passes=False)


@functools.partial(
    pl.kernel,
    out_type=jax.ShapeDtypeStruct((NW, N), jnp.float32),
    mesh=_mesh,
    scratch_types=[
        pltpu.VMEM((104, CHUNK), jnp.int32),
        pltpu.VMEM((N,), jnp.float32),
    ],
    compiler_params=_sc_params,
)
def _deg_kernel(ei_hbm, degp_hbm, dstv, degv):
    c = lax.axis_index("c")
    s = lax.axis_index("s")
    wid = s * NC + c
    half = wid % NC
    pltpu.sync_copy(ei_hbm.at[1, wid // NC, pl.ds(half * 96, 104)], dstv)

    zeros = jnp.zeros((16,), jnp.float32)

    def zero_body(i, carry):
        degv[pl.ds(i * 16, 16)] = zeros
        return carry

    lax.fori_loop(0, N // 16, zero_body, 0)

    ones = jnp.ones((16,), jnp.float32)
    lanes = lax.iota(jnp.int32, 16)

    def add_body_from(base):
        def add_body(i, carry):
            flat = base + i * 16 + lanes
            r = flat // CHUNK
            col = flat - r * CHUNK
            idx = plsc.load_gather(dstv, [r, col])
            plsc.addupdate_scatter(degv, [idx], ones)
            return carry
        return add_body

    @pl.when(half == 0)
    def _():
        lax.fori_loop(0, (104 * CHUNK) // 16, add_body_from(0), 0)

    @pl.when(half == 1)
    def _():
        lax.fori_loop(0, (96 * CHUNK) // 16, add_body_from(8 * CHUNK), 0)

    pltpu.sync_copy(degv, degp_hbm.at[wid])


def _dinv_col(degp):
    ones = jnp.ones((NW, 1), jnp.float32)
    deg = lax.dot_general(degp, ones, (((0,), (0,)), ((), ())),
                          preferred_element_type=jnp.float32) + 1.0
    return lax.rsqrt(deg)


def _front_body(x_ref, wa_ref, wc_ref, degp_ref, ya_ref, yc_ref):
    dv = _dinv_col(degp_ref[...])
    xv = x_ref[...]
    f32 = jnp.float32
    ya_ref[...] = jnp.dot(xv, wa_ref[...], preferred_element_type=f32) * dv
    yc_ref[...] = jnp.dot(xv, wc_ref[...], preferred_element_type=f32) * dv


_front_call = pl.pallas_call(
    _front_body,
    out_shape=[
        jax.ShapeDtypeStruct((N, D), jnp.float32),
        jax.ShapeDtypeStruct((N, D), jnp.float32),
    ],
)


@functools.partial(
    pl.kernel,
    out_type=[
        jax.ShapeDtypeStruct((N, D), jnp.float32),
        jax.ShapeDtypeStruct((N, D), jnp.float32),
    ],
    mesh=_mesh,
    scratch_types=[
        pltpu.VMEM((SLAB, CHUNK), jnp.int32),
        pltpu.VMEM((SLAB, CHUNK), jnp.int32),
        pltpu.VMEM((CHUNK, D), jnp.float32),
        pltpu.VMEM((CHUNK, D), jnp.float32),
        pltpu.VMEM((CHUNK, D), jnp.float32),
        pltpu.VMEM_SHARED((N, D), jnp.float32),
        pltpu.SemaphoreType.DMA,
        pltpu.SemaphoreType.DMA,
        pltpu.SemaphoreType.DMA,
        pltpu.SemaphoreType.DMA,
        pltpu.SemaphoreType.DMA,
        pltpu.SemaphoreType.DMA,
    ],
    compiler_params=_sc_params,
)
def _edge_kernel(ya_hbm, yc_hbm, ei_hbm, acca_hbm, accc_hbm,
                 srcv, dstv, rows0, rows1, rows2, accs,
                 g0, g1, g2, s0, s1, s2):
    c = lax.axis_index("c")
    s = lax.axis_index("s")
    bufs = (rows0, rows1, rows2)
    gsems = (g0, g1, g2)
    ssems = (s0, s1, s2)

    def blocks_copy(src_at, dst_at, sem):
        for j in range((NBLK + NS - 1) // NS):
            g = j * NS + s

            @pl.when(g < NBLK)
            def _():
                pltpu.async_copy(src_at(g), dst_at(g), sem)
        for j in range((NBLK + NS - 1) // NS):
            g = j * NS + s

            @pl.when(g < NBLK)
            def _():
                pltpu.make_async_copy(src_at(g), dst_at(g), sem).wait()

    def run(y_hbm, out_hbm):
        blocks_copy(lambda g: y_hbm.at[pl.ds(g * BLK, BLK)],
                    lambda g: accs.at[pl.ds(g * BLK, BLK)], g0)
        plsc.subcore_barrier()

        def gstart(l, b):
            pltpu.async_copy(y_hbm.at[srcv.at[l]], bufs[b], gsems[b])

        def gwait(b):
            pltpu.make_async_copy(y_hbm.at[srcv.at[0]], bufs[b], gsems[b]).wait()

        def sstart(l, b):
            pltpu.async_copy(bufs[b], accs.at[dstv.at[l]], ssems[b], add=True)

        def swait(b):
            pltpu.make_async_copy(bufs[b], accs.at[dstv.at[0]], ssems[b]).wait()

        for slab in range(NSLABS):
            ph = (slab * SLAB) % NBUF
            pltpu.async_copy(ei_hbm.at[0, s, pl.ds(slab * SLAB, SLAB)], srcv, g0)
            pltpu.async_copy(ei_hbm.at[1, s, pl.ds(slab * SLAB, SLAB)], dstv, g1)
            pltpu.make_async_copy(ei_hbm.at[0, s, pl.ds(slab * SLAB, SLAB)],
                                  srcv, g0).wait()
            pltpu.make_async_copy(ei_hbm.at[1, s, pl.ds(slab * SLAB, SLAB)],
                                  dstv, g1).wait()
            for l in range(NBUF - 1):
                gstart(l, (l + ph) % NBUF)

            def group(m, carry):
                l0 = m * NBUF
                for k in range(NBUF):
                    b = (k + ph) % NBUF
                    gwait(b)
                    sstart(l0 + k, b)
                    if k == 0:
                        @pl.when(m > 0)
                        def _():
                            swait((ph - 1) % NBUF)
                    else:
                        swait((k - 1 + ph) % NBUF)
                    if k == 0:
                        gstart(l0 + k + NBUF - 1, (b + NBUF - 1) % NBUF)
                    else:
                        @pl.when(l0 + k + NBUF - 1 < SLAB)
                        def _():
                            gstart(l0 + k + NBUF - 1, (b + NBUF - 1) % NBUF)
                return carry

            ngroups = SLAB // NBUF
            lax.fori_loop(0, ngroups, group, 0)
            for l in range(ngroups * NBUF, SLAB):
                b = (l + ph) % NBUF
                gwait(b)
                sstart(l, b)
                swait((b + NBUF - 1) % NBUF)
            swait((SLAB - 1 + ph) % NBUF)
        plsc.subcore_barrier()
        blocks_copy(lambda g: accs.at[pl.ds(g * BLK, BLK)],
                    lambda g: out_hbm.at[pl.ds(g * BLK, BLK)], g0)

    @pl.when(c == 0)
    def _():
        run(ya_hbm, acca_hbm)

    @pl.when(c == 1)
    def _():
        run(yc_hbm, accc_hbm)


def _softplus(v):
    return jnp.maximum(v, 0.0) + jnp.log1p(jnp.exp(-jnp.abs(v)))


def _epi_body(acca_ref, accc_ref, degp_ref, x_ref,
              bca_ref, w1a_ref, b1a_ref, w2a_ref, b2a_ref, w3a_ref, b3a_ref,
              bcc_ref, w1c_ref, b1c_ref, w2c_ref, b2c_ref, w3c_ref, b3c_ref,
              conc_ref, probs_ref, val_ref):
    dv = _dinv_col(degp_ref[...])
    xv = x_ref[...]
    f32 = jnp.float32

    ha = jnp.maximum(dv * acca_ref[...] + bca_ref[...], 0.0)
    xa = ha + xv
    t = jnp.maximum(jnp.dot(xa, w1a_ref[...], preferred_element_type=f32)
                    + b1a_ref[...], 0.0)
    t = jnp.maximum(jnp.dot(t, w2a_ref[...], preferred_element_type=f32)
                    + b2a_ref[...], 0.0)
    ao = (lax.dot_general(w3a_ref[...], t, (((0,), (1,)), ((), ())),
                          preferred_element_type=f32)
          + b3a_ref[...][:, None])
    conc_ref[...] = _softplus(ao[0:1, :]) + 1e-20
    p2 = _softplus(ao[1:2, :])
    probs_ref[...] = p2 / jnp.sum(p2)

    hc = jnp.maximum(dv * accc_ref[...] + bcc_ref[...], 0.0)
    xc = jnp.sum(hc + xv, axis=0, keepdims=True)
    u = jnp.maximum(jnp.dot(xc, w1c_ref[...], preferred_element_type=f32)
                    + b1c_ref[...], 0.0)
    u = jnp.maximum(jnp.dot(u, w2c_ref[...], preferred_element_type=f32)
                    + b2c_ref[...], 0.0)
    val_ref[...] = jnp.dot(u, w3c_ref[...], preferred_element_type=f32) + b3c_ref[...]


_epi_call = pl.pallas_call(
    _epi_body,
    out_shape=[
        jax.ShapeDtypeStruct((1, N), jnp.float32),
        jax.ShapeDtypeStruct((1, N), jnp.float32),
        jax.ShapeDtypeStruct((1, 2), jnp.float32),
    ],
)


def kernel(x, edge_index, Wc_a, bc_a, W1_a, b1_a, W2_a, b2_a, W3_a, b3_a,
           Wc_c, bc_c, W1_c, b1_c, W2_c, b2_c, W3_c, b3_c):
    ei4 = edge_index.reshape(2, NS, NCHUNKS, CHUNK)
    degp = _deg_kernel(ei4)
    ya, yc = _front_call(x, Wc_a, Wc_c, degp)

    acca, accc = _edge_kernel(ya, yc, ei4)

    conc, probs, value = _epi_call(
        acca, accc, degp, x,
        bc_a, W1_a, b1_a, W2_a, b2_a, W3_a, b3_a,
        bc_c, W1_c, b1_c, W2_c, b2_c, W3_c, b3_c,
    )
    return conc.reshape(-1), value.reshape(-1), probs.reshape(-1)


# speedup vs baseline: 1.0856x; 1.0294x over previous
import functools

import jax
import jax.numpy as jnp
from jax import lax
from jax.experimental import pallas as pl
from jax.experimental.pallas import tpu as pltpu
from jax.experimental.pallas import tpu_sc as plsc

N = 10000
E = 320000
D = 128

NC = 2
NS = 16
NW = NC * NS

EDGES_PER_TILE = E // NS
CHUNK = 100
NCHUNKS = EDGES_PER_TILE // CHUNK
BLK = 80
NBLK = N // BLK
SLAB = 40
NSLABS = NCHUNKS // SLAB
NBUF = 3

_mesh = plsc.VectorSubcoreMesh(core_axis_name="c", subcore_axis_name="s")
_sc_params = pltpu.CompilerParams(needs_layout_passes=False)


@functools.partial(
    pl.kernel,
    out_type=jax.ShapeDtypeStruct((NW, N), jnp.float32),
    mesh=_mesh,
    scratch_types=[
        pltpu.VMEM((2, 10240), jnp.int32),
        pltpu.VMEM((N,), jnp.float32),
    ],
    compiler_params=_sc_params,
)
def _deg_kernel(ei_hbm, degp_hbm, ev, degv):
    c = lax.axis_index("c")
    s = lax.axis_index("s")
    wid = s * NC + c

    zeros = jnp.zeros((16,), jnp.float32)

    def zero_body(i, carry):
        degv[pl.ds(i * 16, 16)] = zeros
        return carry

    lax.fori_loop(0, N // 16, zero_body, 0)

    ones = jnp.ones((16,), jnp.float32)

    def add_body(i, carry):
        idx = ev[1, pl.ds(i * 16, 16)]
        plsc.addupdate_scatter(degv, [idx], ones)
        return carry

    @pl.when(wid < NW - 1)
    def _():
        pltpu.sync_copy(ei_hbm.at[:, pl.ds(wid * 10240, 10240)], ev)
        lax.fori_loop(0, 10240 // 16, add_body, 0)

    @pl.when(wid == NW - 1)
    def _():
        pltpu.sync_copy(ei_hbm.at[:, pl.ds((NW - 1) * 10240, 2560)],
                        ev.at[:, pl.ds(0, 2560)])
        lax.fori_loop(0, 2560 // 16, add_body, 0)

    pltpu.sync_copy(degv, degp_hbm.at[wid])


def _dinv_col(degp):
    ones = jnp.ones((NW, 1), jnp.float32)
    deg = lax.dot_general(degp, ones, (((0,), (0,)), ((), ())),
                          preferred_element_type=jnp.float32) + 1.0
    return lax.rsqrt(deg)


def _front_body(x_ref, wa_ref, wc_ref, degp_ref, ya_ref, yc_ref):
    dv = _dinv_col(degp_ref[...])
    xv = x_ref[...]
    f32 = jnp.float32
    ya_ref[...] = jnp.dot(xv, wa_ref[...], preferred_element_type=f32) * dv
    yc_ref[...] = jnp.dot(xv, wc_ref[...], preferred_element_type=f32) * dv


_front_call = pl.pallas_call(
    _front_body,
    out_shape=[
        jax.ShapeDtypeStruct((N, D), jnp.float32),
        jax.ShapeDtypeStruct((N, D), jnp.float32),
    ],
)


@functools.partial(
    pl.kernel,
    out_type=[
        jax.ShapeDtypeStruct((N, D), jnp.float32),
        jax.ShapeDtypeStruct((N, D), jnp.float32),
    ],
    mesh=_mesh,
    scratch_types=[
        pltpu.VMEM((SLAB, CHUNK), jnp.int32),
        pltpu.VMEM((SLAB, CHUNK), jnp.int32),
        pltpu.VMEM((CHUNK, D), jnp.float32),
        pltpu.VMEM((CHUNK, D), jnp.float32),
        pltpu.VMEM((CHUNK, D), jnp.float32),
        pltpu.VMEM_SHARED((N, D), jnp.float32),
        pltpu.SemaphoreType.DMA,
        pltpu.SemaphoreType.DMA,
        pltpu.SemaphoreType.DMA,
        pltpu.SemaphoreType.DMA,
        pltpu.SemaphoreType.DMA,
        pltpu.SemaphoreType.DMA,
    ],
    compiler_params=_sc_params,
)
def _edge_kernel(ya_hbm, yc_hbm, ei_hbm, acca_hbm, accc_hbm,
                 srcv, dstv, rows0, rows1, rows2, accs,
                 g0, g1, g2, s0, s1, s2):
    c = lax.axis_index("c")
    s = lax.axis_index("s")
    bufs = (rows0, rows1, rows2)
    gsems = (g0, g1, g2)
    ssems = (s0, s1, s2)

    def blocks_copy(src_at, dst_at, sem):
        for j in range((NBLK + NS - 1) // NS):
            g = j * NS + s

            @pl.when(g < NBLK)
            def _():
                pltpu.async_copy(src_at(g), dst_at(g), sem)
        for j in range((NBLK + NS - 1) // NS):
            g = j * NS + s

            @pl.when(g < NBLK)
            def _():
                pltpu.make_async_copy(src_at(g), dst_at(g), sem).wait()

    def run(y_hbm, out_hbm):
        blocks_copy(lambda g: y_hbm.at[pl.ds(g * BLK, BLK)],
                    lambda g: accs.at[pl.ds(g * BLK, BLK)], g0)
        plsc.subcore_barrier()

        def gstart(l, b):
            pltpu.async_copy(y_hbm.at[srcv.at[l]], bufs[b], gsems[b])

        def gwait(b):
            pltpu.make_async_copy(y_hbm.at[srcv.at[0]], bufs[b], gsems[b]).wait()

        def sstart(l, b):
            pltpu.async_copy(bufs[b], accs.at[dstv.at[l]], ssems[b], add=True)

        def swait(b):
            pltpu.make_async_copy(bufs[b], accs.at[dstv.at[0]], ssems[b]).wait()

        for slab in range(NSLABS):
            ph = (slab * SLAB) % NBUF
            pltpu.async_copy(ei_hbm.at[0, s, pl.ds(slab * SLAB, SLAB)], srcv, g0)
            pltpu.async_copy(ei_hbm.at[1, s, pl.ds(slab * SLAB, SLAB)], dstv, g1)
            pltpu.make_async_copy(ei_hbm.at[0, s, pl.ds(slab * SLAB, SLAB)],
                                  srcv, g0).wait()
            pltpu.make_async_copy(ei_hbm.at[1, s, pl.ds(slab * SLAB, SLAB)],
                                  dstv, g1).wait()
            for l in range(NBUF - 1):
                gstart(l, (l + ph) % NBUF)

            def group(m, carry):
                l0 = m * NBUF
                for k in range(NBUF):
                    b = (k + ph) % NBUF
                    gwait(b)
                    sstart(l0 + k, b)
                    if k == 0:
                        @pl.when(m > 0)
                        def _():
                            swait((ph - 1) % NBUF)
                    else:
                        swait((k - 1 + ph) % NBUF)
                    if k == 0:
                        gstart(l0 + k + NBUF - 1, (b + NBUF - 1) % NBUF)
                    else:
                        @pl.when(l0 + k + NBUF - 1 < SLAB)
                        def _():
                            gstart(l0 + k + NBUF - 1, (b + NBUF - 1) % NBUF)
                return carry

            ngroups = SLAB // NBUF
            lax.fori_loop(0, ngroups, group, 0)
            for l in range(ngroups * NBUF, SLAB):
                b = (l + ph) % NBUF
                gwait(b)
                sstart(l, b)
                swait((b + NBUF - 1) % NBUF)
            swait((SLAB - 1 + ph) % NBUF)
        plsc.subcore_barrier()
        blocks_copy(lambda g: accs.at[pl.ds(g * BLK, BLK)],
                    lambda g: out_hbm.at[pl.ds(g * BLK, BLK)], g0)

    @pl.when(c == 0)
    def _():
        run(ya_hbm, acca_hbm)

    @pl.when(c == 1)
    def _():
        run(yc_hbm, accc_hbm)


def _softplus(v):
    return jnp.maximum(v, 0.0) + jnp.log1p(jnp.exp(-jnp.abs(v)))


def _epi_body(acca_ref, accc_ref, degp_ref, x_ref,
              bca_ref, w1a_ref, b1a_ref, w2a_ref, b2a_ref, w3a_ref, b3a_ref,
              bcc_ref, w1c_ref, b1c_ref, w2c_ref, b2c_ref, w3c_ref, b3c_ref,
              conc_ref, probs_ref, val_ref):
    dv = _dinv_col(degp_ref[...])
    xv = x_ref[...]
    f32 = jnp.float32

    ha = jnp.maximum(dv * acca_ref[...] + bca_ref[...], 0.0)
    xa = ha + xv
    t = jnp.maximum(jnp.dot(xa, w1a_ref[...], preferred_element_type=f32)
                    + b1a_ref[...], 0.0)
    t = jnp.maximum(jnp.dot(t, w2a_ref[...], preferred_element_type=f32)
                    + b2a_ref[...], 0.0)
    ao = (lax.dot_general(w3a_ref[...], t, (((0,), (1,)), ((), ())),
                          preferred_element_type=f32)
          + b3a_ref[...][:, None])
    conc_ref[...] = _softplus(ao[0:1, :]) + 1e-20
    p2 = _softplus(ao[1:2, :])
    probs_ref[...] = p2 / jnp.sum(p2)

    hc = jnp.maximum(dv * accc_ref[...] + bcc_ref[...], 0.0)
    xc = jnp.sum(hc + xv, axis=0, keepdims=True)
    u = jnp.maximum(jnp.dot(xc, w1c_ref[...], preferred_element_type=f32)
                    + b1c_ref[...], 0.0)
    u = jnp.maximum(jnp.dot(u, w2c_ref[...], preferred_element_type=f32)
                    + b2c_ref[...], 0.0)
    val_ref[...] = jnp.dot(u, w3c_ref[...], preferred_element_type=f32) + b3c_ref[...]


_epi_call = pl.pallas_call(
    _epi_body,
    out_shape=[
        jax.ShapeDtypeStruct((1, N), jnp.float32),
        jax.ShapeDtypeStruct((1, N), jnp.float32),
        jax.ShapeDtypeStruct((1, 2), jnp.float32),
    ],
)


def kernel(x, edge_index, Wc_a, bc_a, W1_a, b1_a, W2_a, b2_a, W3_a, b3_a,
           Wc_c, bc_c, W1_c, b1_c, W2_c, b2_c, W3_c, b3_c):
    ei4 = edge_index.reshape(2, NS, NCHUNKS, CHUNK)
    degp = _deg_kernel(edge_index)
    ya, yc = _front_call(x, Wc_a, Wc_c, degp)

    acca, accc = _edge_kernel(ya, yc, ei4)

    conc, probs, value = _epi_call(
        acca, accc, degp, x,
        bc_a, W1_a, b1_a, W2_a, b2_a, W3_a, b3_a,
        bc_c, W1_c, b1_c, W2_c, b2_c, W3_c, b3_c,
    )
    return conc.reshape(-1), value.reshape(-1), probs.reshape(-1)


# speedup vs baseline: 1.1002x; 1.0134x over previous
import functools

import jax
import jax.numpy as jnp
from jax import lax
from jax.experimental import pallas as pl
from jax.experimental.pallas import tpu as pltpu
from jax.experimental.pallas import tpu_sc as plsc

N = 10000
E = 320000
D = 128

NC = 2
NS = 16
NW = NC * NS

EDGES_PER_TILE = E // NS
CHUNK = 100
NCHUNKS = EDGES_PER_TILE // CHUNK
BLK = 80
NBLK = N // BLK
SLAB = 40
NSLABS = NCHUNKS // SLAB
NBUF = 3

_mesh = plsc.VectorSubcoreMesh(core_axis_name="c", subcore_axis_name="s")
_sc_params = pltpu.CompilerParams(needs_layout_passes=False)


@functools.partial(
    pl.kernel,
    out_type=jax.ShapeDtypeStruct((NW, N), jnp.float32),
    mesh=_mesh,
    scratch_types=[
        pltpu.VMEM((2, 10240), jnp.int32),
        pltpu.VMEM((N,), jnp.float32),
    ],
    compiler_params=_sc_params,
)
def _deg_kernel(ei_hbm, degp_hbm, ev, degv):
    c = lax.axis_index("c")
    s = lax.axis_index("s")
    wid = s * NC + c

    zeros = jnp.zeros((16,), jnp.float32)

    def zero_body(i, carry):
        degv[pl.ds(i * 16, 16)] = zeros
        return carry

    lax.fori_loop(0, N // 16, zero_body, 0)

    ones = jnp.ones((16,), jnp.float32)

    def add_body(i, carry):
        idx = ev[1, pl.ds(i * 16, 16)]
        plsc.addupdate_scatter(degv, [idx], ones)
        return carry

    @pl.when(wid < NW - 1)
    def _():
        pltpu.sync_copy(ei_hbm.at[:, pl.ds(wid * 10240, 10240)], ev)
        lax.fori_loop(0, 10240 // 16, add_body, 0)

    @pl.when(wid == NW - 1)
    def _():
        pltpu.sync_copy(ei_hbm.at[:, pl.ds((NW - 1) * 10240, 2560)],
                        ev.at[:, pl.ds(0, 2560)])
        lax.fori_loop(0, 2560 // 16, add_body, 0)

    pltpu.sync_copy(degv, degp_hbm.at[wid])


def _dinv_col(degp):
    ones = jnp.ones((NW, 1), jnp.float32)
    deg = lax.dot_general(degp, ones, (((0,), (0,)), ((), ())),
                          preferred_element_type=jnp.float32) + 1.0
    return lax.rsqrt(deg)


def _front_body(x_ref, wa_ref, wc_ref, degp_ref, ya_ref, yc_ref):
    dv = _dinv_col(degp_ref[...])
    xv = x_ref[...]
    f32 = jnp.float32
    ya_ref[...] = jnp.dot(xv, wa_ref[...], preferred_element_type=f32) * dv
    yc_ref[...] = jnp.dot(xv, wc_ref[...], preferred_element_type=f32) * dv


_front_call = pl.pallas_call(
    _front_body,
    out_shape=[
        jax.ShapeDtypeStruct((N, D), jnp.float32),
        jax.ShapeDtypeStruct((N, D), jnp.float32),
    ],
)


@functools.partial(
    pl.kernel,
    out_type=[
        jax.ShapeDtypeStruct((N, D), jnp.float32),
        jax.ShapeDtypeStruct((N, D), jnp.float32),
    ],
    mesh=_mesh,
    scratch_types=[
        pltpu.VMEM((SLAB, CHUNK), jnp.int32),
        pltpu.VMEM((SLAB, CHUNK), jnp.int32),
        pltpu.VMEM((CHUNK, D), jnp.float32),
        pltpu.VMEM((CHUNK, D), jnp.float32),
        pltpu.VMEM((CHUNK, D), jnp.float32),
        pltpu.VMEM_SHARED((N, D), jnp.float32),
        pltpu.SemaphoreType.DMA,
        pltpu.SemaphoreType.DMA,
        pltpu.SemaphoreType.DMA,
        pltpu.SemaphoreType.DMA,
        pltpu.SemaphoreType.DMA,
        pltpu.SemaphoreType.DMA,
    ],
    compiler_params=_sc_params,
)
def _edge_kernel(ya_hbm, yc_hbm, ei_hbm, acca_hbm, accc_hbm,
                 srcv, dstv, rows0, rows1, rows2, accs,
                 g0, g1, g2, s0, s1, s2):
    c = lax.axis_index("c")
    s = lax.axis_index("s")
    bufs = (rows0, rows1, rows2)
    gsems = (g0, g1, g2)
    ssems = (s0, s1, s2)

    def blocks_copy(src_at, dst_at, sem):
        for j in range((NBLK + NS - 1) // NS):
            g = j * NS + s

            @pl.when(g < NBLK)
            def _():
                pltpu.async_copy(src_at(g), dst_at(g), sem)
        for j in range((NBLK + NS - 1) // NS):
            g = j * NS + s

            @pl.when(g < NBLK)
            def _():
                pltpu.make_async_copy(src_at(g), dst_at(g), sem).wait()

    def run(y_hbm, out_hbm):
        blocks_copy(lambda g: y_hbm.at[pl.ds(g * BLK, BLK)],
                    lambda g: accs.at[pl.ds(g * BLK, BLK)], g0)
        plsc.subcore_barrier()

        def gstart(l, b):
            pltpu.async_copy(y_hbm.at[srcv.at[l]], bufs[b], gsems[b])

        def gwait(b):
            pltpu.make_async_copy(y_hbm.at[srcv.at[0]], bufs[b], gsems[b]).wait()

        def sstart(l, b):
            pltpu.async_copy(bufs[b], accs.at[dstv.at[l]], ssems[b], add=True)

        def swait(b):
            pltpu.make_async_copy(bufs[b], accs.at[dstv.at[0]], ssems[b]).wait()

        for slab in range(NSLABS):
            ph = (slab * SLAB) % NBUF
            pltpu.async_copy(ei_hbm.at[0, s, pl.ds(slab * SLAB, SLAB)], srcv, g0)
            pltpu.async_copy(ei_hbm.at[1, s, pl.ds(slab * SLAB, SLAB)], dstv, g1)
            pltpu.make_async_copy(ei_hbm.at[0, s, pl.ds(slab * SLAB, SLAB)],
                                  srcv, g0).wait()
            pltpu.make_async_copy(ei_hbm.at[1, s, pl.ds(slab * SLAB, SLAB)],
                                  dstv, g1).wait()
            for l in range(NBUF - 1):
                gstart(l, (l + ph) % NBUF)

            def group(m, carry):
                l0 = m * NBUF
                for k in range(NBUF):
                    b = (k + ph) % NBUF
                    gwait(b)
                    sstart(l0 + k, b)
                    if k == 0:
                        @pl.when(m > 0)
                        def _():
                            swait((ph - 1) % NBUF)
                    else:
                        swait((k - 1 + ph) % NBUF)
                    if k == 0:
                        gstart(l0 + k + NBUF - 1, (b + NBUF - 1) % NBUF)
                    else:
                        @pl.when(l0 + k + NBUF - 1 < SLAB)
                        def _():
                            gstart(l0 + k + NBUF - 1, (b + NBUF - 1) % NBUF)
                return carry

            ngroups = SLAB // NBUF
            lax.fori_loop(0, ngroups, group, 0)
            for l in range(ngroups * NBUF, SLAB):
                b = (l + ph) % NBUF
                gwait(b)
                sstart(l, b)
                swait((b + NBUF - 1) % NBUF)
            swait((SLAB - 1 + ph) % NBUF)
        plsc.subcore_barrier()
        blocks_copy(lambda g: accs.at[pl.ds(g * BLK, BLK)],
                    lambda g: out_hbm.at[pl.ds(g * BLK, BLK)], g0)

    @pl.when(c == 0)
    def _():
        run(ya_hbm, acca_hbm)

    @pl.when(c == 1)
    def _():
        run(yc_hbm, accc_hbm)


def _softplus(v):
    return jnp.maximum(v, 0.0) + jnp.log1p(jnp.exp(-jnp.abs(v)))


def _epi_body(acca_ref, accc_ref, degp_ref, x_ref,
              bca_ref, w1a_ref, b1a_ref, w2a_ref, b2a_ref, w3a_ref, b3a_ref,
              bcc_ref, w1c_ref, b1c_ref, w2c_ref, b2c_ref, w3c_ref, b3c_ref,
              conc_ref, probs_ref, val_ref):
    dv = _dinv_col(degp_ref[...])
    xv = x_ref[...]
    f32 = jnp.float32

    ha = jnp.maximum(dv * acca_ref[...] + bca_ref[...], 0.0)
    xa = ha + xv
    t = jnp.maximum(jnp.dot(xa, w1a_ref[...], preferred_element_type=f32)
                    + b1a_ref[...], 0.0)
    t = jnp.maximum(jnp.dot(t, w2a_ref[...], preferred_element_type=f32)
                    + b2a_ref[...], 0.0)
    ao = (lax.dot_general(w3a_ref[...], t, (((0,), (1,)), ((), ())),
                          preferred_element_type=f32)
          + b3a_ref[...][:, None])
    conc_ref[...] = jnp.reshape(_softplus(ao[0:1, :]) + 1e-20, (N,))
    p2 = _softplus(ao[1:2, :])
    probs_ref[...] = jnp.reshape(p2 / jnp.sum(p2), (N,))

    hc = jnp.maximum(dv * accc_ref[...] + bcc_ref[...], 0.0)
    xc = jnp.sum(hc + xv, axis=0, keepdims=True)
    u = jnp.maximum(jnp.dot(xc, w1c_ref[...], preferred_element_type=f32)
                    + b1c_ref[...], 0.0)
    u = jnp.maximum(jnp.dot(u, w2c_ref[...], preferred_element_type=f32)
                    + b2c_ref[...], 0.0)
    val_ref[...] = jnp.dot(u, w3c_ref[...], preferred_element_type=f32) + b3c_ref[...]


_epi_call = pl.pallas_call(
    _epi_body,
    out_shape=[
        jax.ShapeDtypeStruct((N,), jnp.float32),
        jax.ShapeDtypeStruct((N,), jnp.float32),
        jax.ShapeDtypeStruct((1, 2), jnp.float32),
    ],
)


def kernel(x, edge_index, Wc_a, bc_a, W1_a, b1_a, W2_a, b2_a, W3_a, b3_a,
           Wc_c, bc_c, W1_c, b1_c, W2_c, b2_c, W3_c, b3_c):
    ei4 = edge_index.reshape(2, NS, NCHUNKS, CHUNK)
    degp = _deg_kernel(edge_index)
    ya, yc = _front_call(x, Wc_a, Wc_c, degp)

    acca, accc = _edge_kernel(ya, yc, ei4)

    conc, probs, value = _epi_call(
        acca, accc, degp, x,
        bc_a, W1_a, b1_a, W2_a, b2_a, W3_a, b3_a,
        bc_c, W1_c, b1_c, W2_c, b2_c, W3_c, b3_c,
    )
    return conc.reshape(-1), value.reshape(-1), probs.reshape(-1)


# speedup vs baseline: 1.1090x; 1.0080x over previous
import functools

import jax
import jax.numpy as jnp
from jax import lax
from jax.experimental import pallas as pl
from jax.experimental.pallas import tpu as pltpu
from jax.experimental.pallas import tpu_sc as plsc

N = 10000
E = 320000
D = 128

NC = 2
NS = 16
NW = NC * NS

EDGES_PER_TILE = E // NS
CHUNK = 100
NCHUNKS = EDGES_PER_TILE // CHUNK
BLK = 80
NBLK = N // BLK
SLAB = 40
NSLABS = NCHUNKS // SLAB
NBUF = 3

_mesh = plsc.VectorSubcoreMesh(core_axis_name="c", subcore_axis_name="s")
_sc_params = pltpu.CompilerParams(needs_layout_passes=False)


@functools.partial(
    pl.kernel,
    out_type=jax.ShapeDtypeStruct((NW, N), jnp.float32),
    mesh=_mesh,
    scratch_types=[
        pltpu.VMEM((2, 10240), jnp.int32),
        pltpu.VMEM((N,), jnp.float32),
    ],
    compiler_params=_sc_params,
)
def _deg_kernel(ei_hbm, degp_hbm, ev, degv):
    c = lax.axis_index("c")
    s = lax.axis_index("s")
    wid = s * NC + c

    zeros = jnp.zeros((16,), jnp.float32)

    def zero_body(i, carry):
        degv[pl.ds(i * 16, 16)] = zeros
        return carry

    lax.fori_loop(0, N // 16, zero_body, 0)

    ones = jnp.ones((16,), jnp.float32)

    def add_body(i, carry):
        idx = ev[1, pl.ds(i * 16, 16)]
        plsc.addupdate_scatter(degv, [idx], ones)
        return carry

    @pl.when(wid < NW - 1)
    def _():
        pltpu.sync_copy(ei_hbm.at[:, pl.ds(wid * 10240, 10240)], ev)
        lax.fori_loop(0, 10240 // 16, add_body, 0)

    @pl.when(wid == NW - 1)
    def _():
        pltpu.sync_copy(ei_hbm.at[:, pl.ds((NW - 1) * 10240, 2560)],
                        ev.at[:, pl.ds(0, 2560)])
        lax.fori_loop(0, 2560 // 16, add_body, 0)

    pltpu.sync_copy(degv, degp_hbm.at[wid])


def _dinv_col(degp):
    ones = jnp.ones((NW, 1), jnp.float32)
    deg = lax.dot_general(degp, ones, (((0,), (0,)), ((), ())),
                          preferred_element_type=jnp.float32) + 1.0
    return lax.rsqrt(deg)


def _front_body(x_ref, wa_ref, wc_ref, degp_ref, ya_ref, yc_ref):
    dv = _dinv_col(degp_ref[...])
    xv = x_ref[...]
    f32 = jnp.float32
    ya_ref[...] = jnp.dot(xv, wa_ref[...], preferred_element_type=f32) * dv
    yc_ref[...] = jnp.dot(xv, wc_ref[...], preferred_element_type=f32) * dv


_front_call = pl.pallas_call(
    _front_body,
    out_shape=[
        jax.ShapeDtypeStruct((N, D), jnp.float32),
        jax.ShapeDtypeStruct((N, D), jnp.float32),
    ],
)


@functools.partial(
    pl.kernel,
    out_type=[
        jax.ShapeDtypeStruct((N, D), jnp.float32),
        jax.ShapeDtypeStruct((N, D), jnp.float32),
    ],
    mesh=_mesh,
    scratch_types=[
        pltpu.VMEM((SLAB, CHUNK), jnp.int32),
        pltpu.VMEM((SLAB, CHUNK), jnp.int32),
        pltpu.VMEM((CHUNK, D), jnp.float32),
        pltpu.VMEM((CHUNK, D), jnp.float32),
        pltpu.VMEM((CHUNK, D), jnp.float32),
        pltpu.VMEM_SHARED((N, D), jnp.float32),
        pltpu.SemaphoreType.DMA,
        pltpu.SemaphoreType.DMA,
        pltpu.SemaphoreType.DMA,
        pltpu.SemaphoreType.DMA,
        pltpu.SemaphoreType.DMA,
        pltpu.SemaphoreType.DMA,
        pltpu.SemaphoreType.DMA,
    ],
    compiler_params=_sc_params,
)
def _edge_kernel(ya_hbm, yc_hbm, ei_hbm, acca_hbm, accc_hbm,
                 srcv, dstv, rows0, rows1, rows2, accs,
                 g0, g1, g2, s0, s1, s2, isem):
    c = lax.axis_index("c")
    s = lax.axis_index("s")
    bufs = (rows0, rows1, rows2)
    gsems = (g0, g1, g2)
    ssems = (s0, s1, s2)

    def blocks_issue(src_at, dst_at, sem):
        for j in range((NBLK + NS - 1) // NS):
            g = j * NS + s

            @pl.when(g < NBLK)
            def _():
                pltpu.async_copy(src_at(g), dst_at(g), sem)

    def blocks_wait(src_at, dst_at, sem):
        for j in range((NBLK + NS - 1) // NS):
            g = j * NS + s

            @pl.when(g < NBLK)
            def _():
                pltpu.make_async_copy(src_at(g), dst_at(g), sem).wait()

    def blocks_copy(src_at, dst_at, sem):
        blocks_issue(src_at, dst_at, sem)
        blocks_wait(src_at, dst_at, sem)

    def run(y_hbm, out_hbm):
        init_src = lambda g: y_hbm.at[pl.ds(g * BLK, BLK)]
        init_dst = lambda g: accs.at[pl.ds(g * BLK, BLK)]
        blocks_issue(init_src, init_dst, isem)

        def gstart(l, b):
            pltpu.async_copy(y_hbm.at[srcv.at[l]], bufs[b], gsems[b])

        def gwait(b):
            pltpu.make_async_copy(y_hbm.at[srcv.at[0]], bufs[b], gsems[b]).wait()

        def sstart(l, b):
            pltpu.async_copy(bufs[b], accs.at[dstv.at[l]], ssems[b], add=True)

        def swait(b):
            pltpu.make_async_copy(bufs[b], accs.at[dstv.at[0]], ssems[b]).wait()

        for slab in range(NSLABS):
            ph = (slab * SLAB) % NBUF
            pltpu.async_copy(ei_hbm.at[0, s, pl.ds(slab * SLAB, SLAB)], srcv, g0)
            pltpu.async_copy(ei_hbm.at[1, s, pl.ds(slab * SLAB, SLAB)], dstv, g1)
            pltpu.make_async_copy(ei_hbm.at[0, s, pl.ds(slab * SLAB, SLAB)],
                                  srcv, g0).wait()
            pltpu.make_async_copy(ei_hbm.at[1, s, pl.ds(slab * SLAB, SLAB)],
                                  dstv, g1).wait()
            for l in range(NBUF - 1):
                gstart(l, (l + ph) % NBUF)

            if slab == 0:
                blocks_wait(init_src, init_dst, isem)
                plsc.subcore_barrier()

            def group(m, carry):
                l0 = m * NBUF
                for k in range(NBUF):
                    b = (k + ph) % NBUF
                    gwait(b)
                    sstart(l0 + k, b)
                    if k == 0:
                        @pl.when(m > 0)
                        def _():
                            swait((ph - 1) % NBUF)
                    else:
                        swait((k - 1 + ph) % NBUF)
                    if k == 0:
                        gstart(l0 + k + NBUF - 1, (b + NBUF - 1) % NBUF)
                    else:
                        @pl.when(l0 + k + NBUF - 1 < SLAB)
                        def _():
                            gstart(l0 + k + NBUF - 1, (b + NBUF - 1) % NBUF)
                return carry

            ngroups = SLAB // NBUF
            lax.fori_loop(0, ngroups, group, 0)
            for l in range(ngroups * NBUF, SLAB):
                b = (l + ph) % NBUF
                gwait(b)
                sstart(l, b)
                swait((b + NBUF - 1) % NBUF)
            swait((SLAB - 1 + ph) % NBUF)
        plsc.subcore_barrier()
        blocks_copy(lambda g: accs.at[pl.ds(g * BLK, BLK)],
                    lambda g: out_hbm.at[pl.ds(g * BLK, BLK)], g0)

    @pl.when(c == 0)
    def _():
        run(ya_hbm, acca_hbm)

    @pl.when(c == 1)
    def _():
        run(yc_hbm, accc_hbm)


def _softplus(v):
    return jnp.maximum(v, 0.0) + jnp.log1p(jnp.exp(-jnp.abs(v)))


def _epi_body(acca_ref, accc_ref, degp_ref, x_ref,
              bca_ref, w1a_ref, b1a_ref, w2a_ref, b2a_ref, w3a_ref, b3a_ref,
              bcc_ref, w1c_ref, b1c_ref, w2c_ref, b2c_ref, w3c_ref, b3c_ref,
              conc_ref, probs_ref, val_ref):
    dv = _dinv_col(degp_ref[...])
    xv = x_ref[...]
    f32 = jnp.float32

    ha = jnp.maximum(dv * acca_ref[...] + bca_ref[...], 0.0)
    xa = ha + xv
    t = jnp.maximum(jnp.dot(xa, w1a_ref[...], preferred_element_type=f32)
                    + b1a_ref[...], 0.0)
    t = jnp.maximum(jnp.dot(t, w2a_ref[...], preferred_element_type=f32)
                    + b2a_ref[...], 0.0)
    ao = (lax.dot_general(w3a_ref[...], t, (((0,), (1,)), ((), ())),
                          preferred_element_type=f32)
          + b3a_ref[...][:, None])
    conc_ref[...] = jnp.reshape(_softplus(ao[0:1, :]) + 1e-20, (N,))
    p2 = _softplus(ao[1:2, :])
    probs_ref[...] = jnp.reshape(p2 / jnp.sum(p2), (N,))

    hc = jnp.maximum(dv * accc_ref[...] + bcc_ref[...], 0.0)
    xc = jnp.sum(hc + xv, axis=0, keepdims=True)
    u = jnp.maximum(jnp.dot(xc, w1c_ref[...], preferred_element_type=f32)
                    + b1c_ref[...], 0.0)
    u = jnp.maximum(jnp.dot(u, w2c_ref[...], preferred_element_type=f32)
                    + b2c_ref[...], 0.0)
    val_ref[...] = jnp.dot(u, w3c_ref[...], preferred_element_type=f32) + b3c_ref[...]


_epi_call = pl.pallas_call(
    _epi_body,
    out_shape=[
        jax.ShapeDtypeStruct((N,), jnp.float32),
        jax.ShapeDtypeStruct((N,), jnp.float32),
        jax.ShapeDtypeStruct((1, 2), jnp.float32),
    ],
)


def kernel(x, edge_index, Wc_a, bc_a, W1_a, b1_a, W2_a, b2_a, W3_a, b3_a,
           Wc_c, bc_c, W1_c, b1_c, W2_c, b2_c, W3_c, b3_c):
    ei4 = edge_index.reshape(2, NS, NCHUNKS, CHUNK)
    degp = _deg_kernel(edge_index)
    ya, yc = _front_call(x, Wc_a, Wc_c, degp)

    acca, accc = _edge_kernel(ya, yc, ei4)

    conc, probs, value = _epi_call(
        acca, accc, degp, x,
        bc_a, W1_a, b1_a, W2_a, b2_a, W3_a, b3_a,
        bc_c, W1_c, b1_c, W2_c, b2_c, W3_c, b3_c,
    )
    return conc.reshape(-1), value.reshape(-1), probs.reshape(-1)
